# baseline (speedup 1.0000x reference)
import functools

import jax
import jax.numpy as jnp
from jax import lax
from jax.experimental import pallas as pl
from jax.experimental.pallas import tpu as pltpu
from jax.experimental.pallas import tpu_sc as plsc

NC = 2
NS = 16
NW = NC * NS

SENT = 4096
SLEN = 50
B = SENT * SLEN
D = 128
BPW = B // NW
NBUF = 8
CHUNK = 80
NCHUNK = BPW // CHUNK

_mesh = plsc.VectorSubcoreMesh(
    core_axis_name="c", subcore_axis_name="s", num_cores=NC, num_subcores=NS
)


@functools.partial(
    pl.kernel,
    out_type=(
        jax.ShapeDtypeStruct((B, D), jnp.float32),
        jax.ShapeDtypeStruct((B, D), jnp.float32),
    ),
    mesh=_mesh,
    scratch_types=[
        pltpu.VMEM((BPW,), jnp.int32),
        pltpu.VMEM((BPW,), jnp.int32),
        [pltpu.VMEM((CHUNK, D), jnp.float32) for _ in range(NBUF)],
        [pltpu.SemaphoreType.DMA for _ in range(NBUF)],
        [pltpu.SemaphoreType.DMA for _ in range(NBUF)],
    ],
)
def _embed_lookup(src_idx, tgt_idx, src_table, tgt_table,
                  src_out, tgt_out, sidx_v, tidx_v, rows, gsem, wsem):
    wid = lax.axis_index("s") * NC + lax.axis_index("c")
    base = wid * BPW

    pltpu.sync_copy(src_idx.at[pl.ds(base, BPW)], sidx_v)
    pltpu.sync_copy(tgt_idx.at[pl.ds(base, BPW)], tidx_v)

    HB = NBUF // 2
    streams = (
        (sidx_v, src_table, src_out, 0),
        (tidx_v, tgt_table, tgt_out, HB),
    )

    def gather(idx_v, table_hbm, j, b):
        return pltpu.make_async_copy(
            table_hbm.at[idx_v.at[pl.ds(j * CHUNK, CHUNK)]], rows[b], gsem[b])

    def writeback(out_hbm, j, b):
        return pltpu.make_async_copy(
            rows[b], out_hbm.at[pl.ds(base + j * CHUNK, CHUNK)], wsem[b])

    for idx_v, table, out, b0 in streams:
        for b in range(HB):
            gather(idx_v, table, b, b0 + b).start()

    @pl.loop(0, NCHUNK - HB, step=HB)
    def _(i):
        for idx_v, table, out, b0 in streams:
            for b in range(HB):
                gather(idx_v, table, i + b, b0 + b).wait()
                writeback(out, i + b, b0 + b).start()
        for idx_v, table, out, b0 in streams:
            for b in range(HB):
                writeback(out, i + b, b0 + b).wait()
                gather(idx_v, table, i + HB + b, b0 + b).start()

    last = NCHUNK - HB
    for idx_v, table, out, b0 in streams:
        for b in range(HB):
            gather(idx_v, table, last + b, b0 + b).wait()
            writeback(out, last + b, b0 + b).start()
    for idx_v, table, out, b0 in streams:
        for b in range(HB):
            writeback(out, last + b, b0 + b).wait()


def kernel(src_indices, tgt_indices, src_table, tgt_table):
    src_t = src_indices.astype(jnp.int32).T.reshape(-1)
    tgt_t = tgt_indices.astype(jnp.int32).T.reshape(-1)
    src_out, tgt_out = _embed_lookup(src_t, tgt_t, src_table, tgt_table)
    src_emb = src_out.reshape(SLEN, SENT, D).transpose(1, 0, 2)
    tgt_emb = tgt_out.reshape(SLEN, SENT, D).transpose(1, 0, 2)
    return (src_emb, tgt_emb)


# speedup vs baseline: 1.0078x; 1.0078x over previous
import functools

import jax
import jax.numpy as jnp
from jax import lax
from jax.experimental import pallas as pl
from jax.experimental.pallas import tpu as pltpu
from jax.experimental.pallas import tpu_sc as plsc

NC = 2
NS = 16
NW = NC * NS

SENT = 4096
SLEN = 50
B = SENT * SLEN
D = 128
BPW = B // NW
NBUF = 10
CHUNK = 64
NCHUNK = BPW // CHUNK

_mesh = plsc.VectorSubcoreMesh(
    core_axis_name="c", subcore_axis_name="s", num_cores=NC, num_subcores=NS
)


@functools.partial(
    pl.kernel,
    out_type=(
        jax.ShapeDtypeStruct((B, D), jnp.float32),
        jax.ShapeDtypeStruct((B, D), jnp.float32),
    ),
    mesh=_mesh,
    scratch_types=[
        pltpu.VMEM((BPW,), jnp.int32),
        pltpu.VMEM((BPW,), jnp.int32),
        [pltpu.VMEM((CHUNK, D), jnp.float32) for _ in range(NBUF)],
        [pltpu.SemaphoreType.DMA for _ in range(NBUF)],
        [pltpu.SemaphoreType.DMA for _ in range(NBUF)],
    ],
)
def _embed_lookup(src_idx, tgt_idx, src_table, tgt_table,
                  src_out, tgt_out, sidx_v, tidx_v, rows, gsem, wsem):
    wid = lax.axis_index("s") * NC + lax.axis_index("c")
    base = wid * BPW

    pltpu.sync_copy(src_idx.at[pl.ds(base, BPW)], sidx_v)
    pltpu.sync_copy(tgt_idx.at[pl.ds(base, BPW)], tidx_v)

    def one_table(idx_v, table_hbm, out_hbm):
        def gather(j, b):
            return pltpu.make_async_copy(
                table_hbm.at[idx_v.at[pl.ds(j * CHUNK, CHUNK)]], rows[b], gsem[b])

        def writeback(j, b):
            return pltpu.make_async_copy(
                rows[b], out_hbm.at[pl.ds(base + j * CHUNK, CHUNK)], wsem[b])

        for b in range(NBUF):
            gather(b, b).start()

        @pl.loop(0, NCHUNK - NBUF, step=NBUF)
        def _(i):
            for b in range(NBUF):
                gather(i + b, b).wait()
                writeback(i + b, b).start()
            for b in range(NBUF):
                writeback(i + b, b).wait()
                gather(i + NBUF + b, b).start()

        last = NCHUNK - NBUF
        for b in range(NBUF):
            gather(last + b, b).wait()
            writeback(last + b, b).start()
        for b in range(NBUF):
            writeback(last + b, b).wait()

    one_table(sidx_v, src_table, src_out)
    one_table(tidx_v, tgt_table, tgt_out)


def kernel(src_indices, tgt_indices, src_table, tgt_table):
    src_t = src_indices.astype(jnp.int32).T.reshape(-1)
    tgt_t = tgt_indices.astype(jnp.int32).T.reshape(-1)
    src_out, tgt_out = _embed_lookup(src_t, tgt_t, src_table, tgt_table)
    src_emb = src_out.reshape(SLEN, SENT, D).transpose(1, 0, 2)
    tgt_emb = tgt_out.reshape(SLEN, SENT, D).transpose(1, 0, 2)
    return (src_emb, tgt_emb)
